# baseline (speedup 1.0000x reference)
import functools

import jax
import jax.numpy as jnp
from jax.experimental import pallas as pl
from jax.experimental.pallas import tpu as pltpu
from jax.experimental.pallas import tpu_sc as plsc

_MESH = plsc.VectorSubcoreMesh(
    core_axis_name="c", subcore_axis_name="s", num_cores=1, num_subcores=1
)


@functools.partial(
    pl.kernel,
    mesh=_MESH,
    out_type=jax.ShapeDtypeStruct((1,), jnp.int32),
    scratch_types=[
        pltpu.VMEM((1,), jnp.int32),
        pltpu.VMEM((1,), jnp.int32),
    ],
)
def _sc_lookup(idx_hbm, table_hbm, out_hbm, idx_v, val_v):
    pltpu.sync_copy(idx_hbm, idx_v)
    pltpu.sync_copy(table_hbm.at[idx_v], val_v)
    pltpu.sync_copy(val_v, out_hbm)


def kernel(y, label_permutation):
    table = label_permutation.astype(jnp.int32)
    idx = jnp.asarray(y, jnp.int32).reshape(1)
    out = _sc_lookup(idx, table)
    return out.reshape(())
